# baseline (speedup 1.0000x reference)
import functools

import jax
import jax.numpy as jnp
from jax import lax
from jax.experimental import pallas as pl
from jax.experimental.pallas import tpu as pltpu
from jax.experimental.pallas import tpu_sc as plsc

N = 10000
D = 128
H = 128
C = 10
G = 64

NTILES = 32
SUBC = 16
CHUNK = 128
SCHUNK = 64
N_ACC = 10240
SUB_ROWS = N_ACC // SUBC

SLOW_CORE = 0
SLOW_FRAC = 0.67

R = 1000
HIGHEST = lax.Precision.HIGHEST
F32 = jnp.float32


def _sc_mesh():
    return plsc.VectorSubcoreMesh(core_axis_name="c", subcore_axis_name="s")


def _make_sc_degree(nchunks):
    @functools.partial(
        pl.kernel,
        out_type=jax.ShapeDtypeStruct((2, N_ACC, H), F32),
        mesh=_sc_mesh(),
        scratch_types=[
            pltpu.VMEM_SHARED((N_ACC, H), F32),
            pltpu.VMEM((nchunks, CHUNK), jnp.int32),
            pltpu.VMEM((CHUNK, H), F32),
        ],
    )
    def deg_kernel(dst_hbm, ones_hbm, zeros_hbm, out_hbm, acc, dst_v, ones_v):
        cid = lax.axis_index("c")
        sid = lax.axis_index("s")
        tid = cid * SUBC + sid
        pltpu.sync_copy(dst_hbm.at[tid], dst_v)
        pltpu.sync_copy(ones_hbm, ones_v)
        pltpu.sync_copy(zeros_hbm, acc.at[pl.ds(sid * SUB_ROWS, SUB_ROWS)])
        plsc.subcore_barrier()

        def body(ci, carry):
            pltpu.sync_copy(ones_v, acc.at[dst_v.at[ci]], add=True)
            return carry

        lax.fori_loop(0, nchunks, body, 0)
        plsc.subcore_barrier()
        pltpu.sync_copy(
            acc.at[pl.ds(sid * SUB_ROWS, SUB_ROWS)],
            out_hbm.at[cid, pl.ds(sid * SUB_ROWS, SUB_ROWS)],
        )

    return deg_kernel


def _make_sc_scatter(c0, c1):
    cmax = max(c0, c1)

    @functools.partial(
        pl.kernel,
        out_type=jax.ShapeDtypeStruct((2, N_ACC, H), F32),
        mesh=_sc_mesh(),
        scratch_types=[
            pltpu.VMEM_SHARED((N_ACC, H), F32),
            pltpu.VMEM((cmax, CHUNK), jnp.int32),
            pltpu.VMEM((cmax, CHUNK), jnp.int32),
            pltpu.VMEM((SCHUNK, H), F32),
            pltpu.VMEM((SCHUNK, H), F32),
            pltpu.SemaphoreType.DMA,
            pltpu.SemaphoreType.DMA,
        ],
    )
    def scat_kernel(h_hbm, src_hbm, dst_hbm, zeros_hbm, out_hbm,
                    acc, src_v, dst_v, rows0, rows1, g0, g1):
        cid = lax.axis_index("c")
        sid = lax.axis_index("s")
        tid = cid * SUBC + sid
        if c0 == c1:
            nchunks = c0
        else:
            nchunks = jnp.where(cid == 0, c0, c1)
        pltpu.sync_copy(src_hbm.at[tid], src_v)
        pltpu.sync_copy(dst_hbm.at[tid], dst_v)
        pltpu.sync_copy(zeros_hbm, acc.at[pl.ds(sid * SUB_ROWS, SUB_ROWS)])
        plsc.subcore_barrier()

        lo = pl.ds(0, SCHUNK)
        hi = pl.ds(SCHUNK, SCHUNK)
        pltpu.async_copy(h_hbm.at[src_v.at[0, lo]], rows0, g0)

        def body(ci, carry):
            cn = jnp.minimum(ci + 1, nchunks - 1)
            pltpu.make_async_copy(h_hbm.at[src_v.at[ci, lo]], rows0, g0).wait()
            pltpu.async_copy(h_hbm.at[src_v.at[ci, hi]], rows1, g1)
            pltpu.sync_copy(rows0, acc.at[dst_v.at[ci, lo]], add=True)
            pltpu.make_async_copy(h_hbm.at[src_v.at[ci, hi]], rows1, g1).wait()
            pltpu.async_copy(h_hbm.at[src_v.at[cn, lo]], rows0, g0)
            pltpu.sync_copy(rows1, acc.at[dst_v.at[ci, hi]], add=True)
            return carry

        lax.fori_loop(0, nchunks, body, 0)
        pltpu.make_async_copy(h_hbm.at[src_v.at[0, lo]], rows0, g0).wait()
        plsc.subcore_barrier()
        pltpu.sync_copy(
            acc.at[pl.ds(sid * SUB_ROWS, SUB_ROWS)],
            out_hbm.at[cid, pl.ds(sid * SUB_ROWS, SUB_ROWS)],
        )

    return scat_kernel


def _tc_first_body(deg_ref, x_ref, w_ref, hp_ref, dinv_ref):
    d = deg_ref[...]
    deg = d[0][:, :1] + d[1][:, :1] + 1.0
    dinv = lax.rsqrt(deg)
    h = jnp.dot(x_ref[...], w_ref[...], precision=HIGHEST)
    hp_ref[...] = dinv * h
    dinv_ref[...] = jnp.broadcast_to(dinv, (R, 16))


def _tc_first(deg2, x, W1):
    return pl.pallas_call(
        _tc_first_body,
        grid=(N // R,),
        in_specs=[
            pl.BlockSpec((2, R, H), lambda i: (0, i, 0)),
            pl.BlockSpec((R, D), lambda i: (i, 0)),
            pl.BlockSpec((D, H), lambda i: (0, 0)),
        ],
        out_specs=[
            pl.BlockSpec((R, H), lambda i: (i, 0)),
            pl.BlockSpec((R, 16), lambda i: (i, 0)),
        ],
        out_shape=[
            jax.ShapeDtypeStruct((N, H), F32),
            jax.ShapeDtypeStruct((N, 16), F32),
        ],
    )(deg2, x, W1)


def _tc_mid_body(s_ref, hp_ref, dinv_ref, b_ref, w_ref, o_ref):
    dinv = dinv_ref[...][:, :1]
    s = s_ref[...]
    agg = s[0] + s[1] + hp_ref[...]
    a = jnp.maximum(dinv * agg + b_ref[...], 0.0)
    o_ref[...] = dinv * jnp.dot(a, w_ref[...], precision=HIGHEST)


def _tc_mid(S, hp, dinv16, b2d, W):
    return pl.pallas_call(
        _tc_mid_body,
        grid=(N // R,),
        in_specs=[
            pl.BlockSpec((2, R, H), lambda i: (0, i, 0)),
            pl.BlockSpec((R, H), lambda i: (i, 0)),
            pl.BlockSpec((R, 16), lambda i: (i, 0)),
            pl.BlockSpec((1, H), lambda i: (0, 0)),
            pl.BlockSpec((H, H), lambda i: (0, 0)),
        ],
        out_specs=pl.BlockSpec((R, H), lambda i: (i, 0)),
        out_shape=jax.ShapeDtypeStruct((N, H), F32),
    )(S, hp, dinv16, b2d, W)


def _tc_final_body(s_ref, hp_ref, dinv_ref, b_ref, batch_ref, wc_ref, bc_ref,
                   o_ref, pooled, cnt):
    i = pl.program_id(0)
    dinv = dinv_ref[...][:, :1]
    s = s_ref[...]
    a = jnp.maximum(dinv * (s[0] + s[1] + hp_ref[...]) + b_ref[...], 0.0)
    bb = batch_ref[0, 0, :]
    gid = lax.broadcasted_iota(jnp.int32, (G, R), 0)
    mask = jnp.where(gid == bb[None, :], 1.0, 0.0).astype(F32)
    pm = jnp.dot(mask, a, precision=HIGHEST)
    cm = jnp.broadcast_to(jnp.sum(mask, axis=1, keepdims=True), (G, H))

    @pl.when(i == 0)
    def _():
        pooled[...] = pm
        cnt[...] = cm

    @pl.when(i > 0)
    def _():
        pooled[...] += pm
        cnt[...] += cm

    @pl.when(i == N // R - 1)
    def _():
        emb = pooled[...] / jnp.maximum(cnt[...], 1.0)
        o_ref[...] = jnp.dot(emb, wc_ref[...], precision=HIGHEST) + bc_ref[...]


def _tc_final(S, hp, dinv16, b2d, batch3, Wcp, bcp):
    return pl.pallas_call(
        _tc_final_body,
        grid=(N // R,),
        in_specs=[
            pl.BlockSpec((2, R, H), lambda i: (0, i, 0)),
            pl.BlockSpec((R, H), lambda i: (i, 0)),
            pl.BlockSpec((R, 16), lambda i: (i, 0)),
            pl.BlockSpec((1, H), lambda i: (0, 0)),
            pl.BlockSpec((1, 1, R), lambda i: (i, 0, 0)),
            pl.BlockSpec((H, 128), lambda i: (0, 0)),
            pl.BlockSpec((1, 128), lambda i: (0, 0)),
        ],
        out_specs=pl.BlockSpec((G, 128), lambda i: (0, 0)),
        out_shape=jax.ShapeDtypeStruct((G, 128), F32),
        scratch_shapes=[
            pltpu.VMEM((G, H), F32),
            pltpu.VMEM((G, H), F32),
        ],
    )(S, hp, dinv16, b2d, batch3, Wcp, bcp)


def _split_uneven(v, fill, c0, c1):
    cmax = max(c0, c1)
    cap0 = SUBC * c0 * CHUNK
    cap1 = SUBC * c1 * CHUNK
    pad = cap0 + cap1 - v.shape[0]
    vp = jnp.concatenate([v, jnp.full((pad,), fill, jnp.int32)])
    p0 = vp[:cap0].reshape(SUBC, c0, CHUNK)
    p1 = vp[cap0:].reshape(SUBC, c1, CHUNK)
    p0 = jnp.pad(p0, ((0, 0), (0, cmax - c0), (0, 0)), constant_values=fill)
    p1 = jnp.pad(p1, ((0, 0), (0, cmax - c1), (0, 0)), constant_values=fill)
    return jnp.concatenate([p0, p1], axis=0)


def kernel(x, edge_index, batch, W1, b1, W2, b2, W3, b3, Wc, bc):
    src = edge_index[0].astype(jnp.int32)
    dst = edge_index[1].astype(jnp.int32)
    E = src.shape[0]
    ept = NTILES * CHUNK
    nchunks = -(-E // ept)
    e_pad = nchunks * ept
    pad = e_pad - E
    dstp = jnp.concatenate([dst, jnp.full((pad,), N, jnp.int32)]).reshape(
        NTILES, nchunks, CHUNK)
    tot = -(-E // CHUNK)
    c_slow = max(1, round(tot * SLOW_FRAC / SUBC))
    c_fast = -(-(tot - SUBC * c_slow) // SUBC)
    c0, c1 = (c_slow, c_fast) if SLOW_CORE == 0 else (c_fast, c_slow)
    srcp_s = _split_uneven(src, 0, c0, c1)
    dstp_s = _split_uneven(dst, N, c0, c1)

    onesH = jnp.ones((CHUNK, H), F32)
    zerosH = jnp.zeros((SUB_ROWS, H), F32)
    b1r = b1.reshape(1, H)
    b2r = b2.reshape(1, H)
    b3r = b3.reshape(1, H)
    Wcp = jnp.zeros((H, 128), F32).at[:, :C].set(Wc)
    bcp = jnp.zeros((1, 128), F32).at[0, :C].set(bc)
    batch3 = batch.astype(jnp.int32).reshape(N // R, 1, R)

    deg2 = _make_sc_degree(nchunks)(dstp, onesH, zerosH)
    h1p, dinv16 = _tc_first(deg2, x, W1)
    scat = _make_sc_scatter(c0, c1)
    S1 = scat(h1p, srcp_s, dstp_s, zerosH)
    h2p = _tc_mid(S1, h1p, dinv16, b1r, W2)
    S2 = scat(h2p, srcp_s, dstp_s, zerosH)
    h3p = _tc_mid(S2, h2p, dinv16, b2r, W3)
    S3 = scat(h3p, srcp_s, dstp_s, zerosH)
    logits = _tc_final(S3, h3p, dinv16, b3r, batch3, Wcp, bcp)
    return logits[:, :C]


# speedup vs baseline: 1.0202x; 1.0202x over previous
import functools

import jax
import jax.numpy as jnp
from jax import lax
from jax.experimental import pallas as pl
from jax.experimental.pallas import tpu as pltpu
from jax.experimental.pallas import tpu_sc as plsc

N = 10000
D = 128
H = 128
C = 10
G = 64

NTILES = 32
SUBC = 16
CHUNK = 128
SCHUNK = 64
N_ACC = 10240
SUB_ROWS = N_ACC // SUBC

SLOW_CORE = 0
SLOW_FRAC = 0.645

R = 1000
HIGHEST = lax.Precision.HIGHEST
F32 = jnp.float32


def _sc_mesh():
    return plsc.VectorSubcoreMesh(core_axis_name="c", subcore_axis_name="s")


def _make_sc_degree(nchunks):
    @functools.partial(
        pl.kernel,
        out_type=jax.ShapeDtypeStruct((2, N_ACC, H), F32),
        mesh=_sc_mesh(),
        scratch_types=[
            pltpu.VMEM_SHARED((N_ACC, H), F32),
            pltpu.VMEM((nchunks, CHUNK), jnp.int32),
            pltpu.VMEM((CHUNK, H), F32),
        ],
    )
    def deg_kernel(dst_hbm, ones_hbm, zeros_hbm, out_hbm, acc, dst_v, ones_v):
        cid = lax.axis_index("c")
        sid = lax.axis_index("s")
        tid = cid * SUBC + sid
        pltpu.sync_copy(dst_hbm.at[tid], dst_v)
        pltpu.sync_copy(ones_hbm, ones_v)
        pltpu.sync_copy(zeros_hbm, acc.at[pl.ds(sid * SUB_ROWS, SUB_ROWS)])
        plsc.subcore_barrier()

        def body(ci, carry):
            pltpu.sync_copy(ones_v, acc.at[dst_v.at[ci]], add=True)
            return carry

        lax.fori_loop(0, nchunks, body, 0)
        plsc.subcore_barrier()
        pltpu.sync_copy(
            acc.at[pl.ds(sid * SUB_ROWS, SUB_ROWS)],
            out_hbm.at[cid, pl.ds(sid * SUB_ROWS, SUB_ROWS)],
        )

    return deg_kernel


def _make_sc_scatter(c0, c1):
    cmax = max(c0, c1)

    @functools.partial(
        pl.kernel,
        out_type=jax.ShapeDtypeStruct((2, N_ACC, H), F32),
        mesh=_sc_mesh(),
        scratch_types=[
            pltpu.VMEM_SHARED((N_ACC, H), F32),
            pltpu.VMEM((cmax, CHUNK), jnp.int32),
            pltpu.VMEM((cmax, CHUNK), jnp.int32),
            pltpu.VMEM((SCHUNK, H), F32),
            pltpu.VMEM((SCHUNK, H), F32),
            pltpu.SemaphoreType.DMA,
            pltpu.SemaphoreType.DMA,
        ],
    )
    def scat_kernel(h_hbm, src_hbm, dst_hbm, zeros_hbm, out_hbm,
                    acc, src_v, dst_v, rows0, rows1, g0, g1):
        cid = lax.axis_index("c")
        sid = lax.axis_index("s")
        tid = cid * SUBC + sid
        if c0 == c1:
            nchunks = c0
        else:
            nchunks = jnp.where(cid == 0, c0, c1)
        pltpu.sync_copy(src_hbm.at[tid], src_v)
        pltpu.sync_copy(dst_hbm.at[tid], dst_v)
        pltpu.sync_copy(zeros_hbm, acc.at[pl.ds(sid * SUB_ROWS, SUB_ROWS)])
        plsc.subcore_barrier()

        lo = pl.ds(0, SCHUNK)
        hi = pl.ds(SCHUNK, SCHUNK)
        pltpu.async_copy(h_hbm.at[src_v.at[0, lo]], rows0, g0)

        def body(ci, carry):
            cn = jnp.minimum(ci + 1, nchunks - 1)
            pltpu.make_async_copy(h_hbm.at[src_v.at[ci, lo]], rows0, g0).wait()
            pltpu.async_copy(h_hbm.at[src_v.at[ci, hi]], rows1, g1)
            pltpu.sync_copy(rows0, acc.at[dst_v.at[ci, lo]], add=True)
            pltpu.make_async_copy(h_hbm.at[src_v.at[ci, hi]], rows1, g1).wait()
            pltpu.async_copy(h_hbm.at[src_v.at[cn, lo]], rows0, g0)
            pltpu.sync_copy(rows1, acc.at[dst_v.at[ci, hi]], add=True)
            return carry

        lax.fori_loop(0, nchunks, body, 0)
        pltpu.make_async_copy(h_hbm.at[src_v.at[0, lo]], rows0, g0).wait()
        plsc.subcore_barrier()
        pltpu.sync_copy(
            acc.at[pl.ds(sid * SUB_ROWS, SUB_ROWS)],
            out_hbm.at[cid, pl.ds(sid * SUB_ROWS, SUB_ROWS)],
        )

    return scat_kernel


def _tc_first_body(deg_ref, x_ref, w_ref, hp_ref, dinv_ref):
    d = deg_ref[...]
    deg = d[0][:, :1] + d[1][:, :1] + 1.0
    dinv = lax.rsqrt(deg)
    h = jnp.dot(x_ref[...], w_ref[...], precision=HIGHEST)
    hp_ref[...] = dinv * h
    dinv_ref[...] = jnp.broadcast_to(dinv, (R, 16))


def _tc_first(deg2, x, W1):
    return pl.pallas_call(
        _tc_first_body,
        grid=(N // R,),
        in_specs=[
            pl.BlockSpec((2, R, H), lambda i: (0, i, 0)),
            pl.BlockSpec((R, D), lambda i: (i, 0)),
            pl.BlockSpec((D, H), lambda i: (0, 0)),
        ],
        out_specs=[
            pl.BlockSpec((R, H), lambda i: (i, 0)),
            pl.BlockSpec((R, 16), lambda i: (i, 0)),
        ],
        out_shape=[
            jax.ShapeDtypeStruct((N, H), F32),
            jax.ShapeDtypeStruct((N, 16), F32),
        ],
    )(deg2, x, W1)


def _tc_mid_body(s_ref, hp_ref, dinv_ref, b_ref, w_ref, o_ref):
    dinv = dinv_ref[...][:, :1]
    s = s_ref[...]
    agg = s[0] + s[1] + hp_ref[...]
    a = jnp.maximum(dinv * agg + b_ref[...], 0.0)
    o_ref[...] = dinv * jnp.dot(a, w_ref[...], precision=HIGHEST)


def _tc_mid(S, hp, dinv16, b2d, W):
    return pl.pallas_call(
        _tc_mid_body,
        grid=(N // R,),
        in_specs=[
            pl.BlockSpec((2, R, H), lambda i: (0, i, 0)),
            pl.BlockSpec((R, H), lambda i: (i, 0)),
            pl.BlockSpec((R, 16), lambda i: (i, 0)),
            pl.BlockSpec((1, H), lambda i: (0, 0)),
            pl.BlockSpec((H, H), lambda i: (0, 0)),
        ],
        out_specs=pl.BlockSpec((R, H), lambda i: (i, 0)),
        out_shape=jax.ShapeDtypeStruct((N, H), F32),
    )(S, hp, dinv16, b2d, W)


def _tc_final_body(s_ref, hp_ref, dinv_ref, b_ref, batch_ref, wc_ref, bc_ref,
                   o_ref, pooled, cnt):
    i = pl.program_id(0)
    dinv = dinv_ref[...][:, :1]
    s = s_ref[...]
    a = jnp.maximum(dinv * (s[0] + s[1] + hp_ref[...]) + b_ref[...], 0.0)
    bb = batch_ref[0, 0, :]
    gid = lax.broadcasted_iota(jnp.int32, (G, R), 0)
    mask = jnp.where(gid == bb[None, :], 1.0, 0.0).astype(F32)
    pm = jnp.dot(mask, a, precision=HIGHEST)
    cm = jnp.broadcast_to(jnp.sum(mask, axis=1, keepdims=True), (G, H))

    @pl.when(i == 0)
    def _():
        pooled[...] = pm
        cnt[...] = cm

    @pl.when(i > 0)
    def _():
        pooled[...] += pm
        cnt[...] += cm

    @pl.when(i == N // R - 1)
    def _():
        emb = pooled[...] / jnp.maximum(cnt[...], 1.0)
        o_ref[...] = jnp.dot(emb, wc_ref[...], precision=HIGHEST) + bc_ref[...]


def _tc_final(S, hp, dinv16, b2d, batch3, Wcp, bcp):
    return pl.pallas_call(
        _tc_final_body,
        grid=(N // R,),
        in_specs=[
            pl.BlockSpec((2, R, H), lambda i: (0, i, 0)),
            pl.BlockSpec((R, H), lambda i: (i, 0)),
            pl.BlockSpec((R, 16), lambda i: (i, 0)),
            pl.BlockSpec((1, H), lambda i: (0, 0)),
            pl.BlockSpec((1, 1, R), lambda i: (i, 0, 0)),
            pl.BlockSpec((H, 128), lambda i: (0, 0)),
            pl.BlockSpec((1, 128), lambda i: (0, 0)),
        ],
        out_specs=pl.BlockSpec((G, 128), lambda i: (0, 0)),
        out_shape=jax.ShapeDtypeStruct((G, 128), F32),
        scratch_shapes=[
            pltpu.VMEM((G, H), F32),
            pltpu.VMEM((G, H), F32),
        ],
    )(S, hp, dinv16, b2d, batch3, Wcp, bcp)


def _split_uneven(v, fill, c0, c1):
    cmax = max(c0, c1)
    cap0 = SUBC * c0 * CHUNK
    cap1 = SUBC * c1 * CHUNK
    pad = cap0 + cap1 - v.shape[0]
    vp = jnp.concatenate([v, jnp.full((pad,), fill, jnp.int32)])
    p0 = vp[:cap0].reshape(SUBC, c0, CHUNK)
    p1 = vp[cap0:].reshape(SUBC, c1, CHUNK)
    p0 = jnp.pad(p0, ((0, 0), (0, cmax - c0), (0, 0)), constant_values=fill)
    p1 = jnp.pad(p1, ((0, 0), (0, cmax - c1), (0, 0)), constant_values=fill)
    return jnp.concatenate([p0, p1], axis=0)


def kernel(x, edge_index, batch, W1, b1, W2, b2, W3, b3, Wc, bc):
    src = edge_index[0].astype(jnp.int32)
    dst = edge_index[1].astype(jnp.int32)
    E = src.shape[0]
    ept = NTILES * CHUNK
    nchunks = -(-E // ept)
    e_pad = nchunks * ept
    pad = e_pad - E
    dstp = jnp.concatenate([dst, jnp.full((pad,), N, jnp.int32)]).reshape(
        NTILES, nchunks, CHUNK)
    tot = -(-E // CHUNK)
    c_slow = max(1, round(tot * SLOW_FRAC / SUBC))
    c_fast = -(-(tot - SUBC * c_slow) // SUBC)
    c0, c1 = (c_slow, c_fast) if SLOW_CORE == 0 else (c_fast, c_slow)
    srcp_s = _split_uneven(src, 0, c0, c1)
    dstp_s = _split_uneven(dst, N, c0, c1)

    onesH = jnp.ones((CHUNK, H), F32)
    zerosH = jnp.zeros((SUB_ROWS, H), F32)
    b1r = b1.reshape(1, H)
    b2r = b2.reshape(1, H)
    b3r = b3.reshape(1, H)
    Wcp = jnp.zeros((H, 128), F32).at[:, :C].set(Wc)
    bcp = jnp.zeros((1, 128), F32).at[0, :C].set(bc)
    batch3 = batch.astype(jnp.int32).reshape(N // R, 1, R)

    deg2 = _make_sc_degree(nchunks)(dstp, onesH, zerosH)
    h1p, dinv16 = _tc_first(deg2, x, W1)
    scat = _make_sc_scatter(c0, c1)
    S1 = scat(h1p, srcp_s, dstp_s, zerosH)
    h2p = _tc_mid(S1, h1p, dinv16, b1r, W2)
    S2 = scat(h2p, srcp_s, dstp_s, zerosH)
    h3p = _tc_mid(S2, h2p, dinv16, b2r, W3)
    S3 = scat(h3p, srcp_s, dstp_s, zerosH)
    logits = _tc_final(S3, h3p, dinv16, b3r, batch3, Wcp, bcp)
    return logits[:, :C]


# speedup vs baseline: 1.0585x; 1.0376x over previous
import functools

import jax
import jax.numpy as jnp
from jax import lax
from jax.experimental import pallas as pl
from jax.experimental.pallas import tpu as pltpu
from jax.experimental.pallas import tpu_sc as plsc

N = 10000
D = 128
H = 128
C = 10
G = 64

NTILES = 32
SUBC = 16
CHUNK = 128
SCHUNK = 64
N_ACC = 10240
SUB_ROWS = N_ACC // SUBC

SLOW_CORE = 0
SLOW_FRAC = 0.62

R = 1000
HIGHEST = lax.Precision.HIGHEST
F32 = jnp.float32


def _sc_mesh():
    return plsc.VectorSubcoreMesh(core_axis_name="c", subcore_axis_name="s")


def _make_sc_degree(nchunks):
    @functools.partial(
        pl.kernel,
        out_type=jax.ShapeDtypeStruct((2, N_ACC, H), F32),
        mesh=_sc_mesh(),
        scratch_types=[
            pltpu.VMEM_SHARED((N_ACC, H), F32),
            pltpu.VMEM((nchunks, CHUNK), jnp.int32),
            pltpu.VMEM((CHUNK, H), F32),
        ],
    )
    def deg_kernel(dst_hbm, ones_hbm, zeros_hbm, out_hbm, acc, dst_v, ones_v):
        cid = lax.axis_index("c")
        sid = lax.axis_index("s")
        tid = cid * SUBC + sid
        pltpu.sync_copy(dst_hbm.at[tid], dst_v)
        pltpu.sync_copy(ones_hbm, ones_v)
        pltpu.sync_copy(zeros_hbm, acc.at[pl.ds(sid * SUB_ROWS, SUB_ROWS)])
        plsc.subcore_barrier()

        def body(ci, carry):
            pltpu.sync_copy(ones_v, acc.at[dst_v.at[ci]], add=True)
            return carry

        lax.fori_loop(0, nchunks, body, 0)
        plsc.subcore_barrier()
        pltpu.sync_copy(
            acc.at[pl.ds(sid * SUB_ROWS, SUB_ROWS)],
            out_hbm.at[cid, pl.ds(sid * SUB_ROWS, SUB_ROWS)],
        )

    return deg_kernel


def _make_sc_scatter(c0, c1):
    cmax = max(c0, c1)

    @functools.partial(
        pl.kernel,
        out_type=jax.ShapeDtypeStruct((2, N_ACC, H), F32),
        mesh=_sc_mesh(),
        scratch_types=[
            pltpu.VMEM_SHARED((N_ACC, H), F32),
            pltpu.VMEM((cmax, CHUNK), jnp.int32),
            pltpu.VMEM((cmax, CHUNK), jnp.int32),
            pltpu.VMEM((SCHUNK, H), F32),
            pltpu.VMEM((SCHUNK, H), F32),
            pltpu.SemaphoreType.DMA,
            pltpu.SemaphoreType.DMA,
        ],
    )
    def scat_kernel(h_hbm, src_hbm, dst_hbm, zeros_hbm, out_hbm,
                    acc, src_v, dst_v, rows0, rows1, g0, g1):
        cid = lax.axis_index("c")
        sid = lax.axis_index("s")
        tid = cid * SUBC + sid
        if c0 == c1:
            nchunks = c0
        else:
            nchunks = jnp.where(cid == 0, c0, c1)
        pltpu.sync_copy(src_hbm.at[tid], src_v)
        pltpu.sync_copy(dst_hbm.at[tid], dst_v)
        pltpu.sync_copy(zeros_hbm, acc.at[pl.ds(sid * SUB_ROWS, SUB_ROWS)])
        plsc.subcore_barrier()

        lo = pl.ds(0, SCHUNK)
        hi = pl.ds(SCHUNK, SCHUNK)
        pltpu.async_copy(h_hbm.at[src_v.at[0, lo]], rows0, g0)

        def body(ci, carry):
            cn = jnp.minimum(ci + 1, nchunks - 1)
            pltpu.make_async_copy(h_hbm.at[src_v.at[ci, lo]], rows0, g0).wait()
            pltpu.async_copy(h_hbm.at[src_v.at[ci, hi]], rows1, g1)
            pltpu.sync_copy(rows0, acc.at[dst_v.at[ci, lo]], add=True)
            pltpu.make_async_copy(h_hbm.at[src_v.at[ci, hi]], rows1, g1).wait()
            pltpu.async_copy(h_hbm.at[src_v.at[cn, lo]], rows0, g0)
            pltpu.sync_copy(rows1, acc.at[dst_v.at[ci, hi]], add=True)
            return carry

        lax.fori_loop(0, nchunks, body, 0)
        pltpu.make_async_copy(h_hbm.at[src_v.at[0, lo]], rows0, g0).wait()
        plsc.subcore_barrier()
        pltpu.sync_copy(
            acc.at[pl.ds(sid * SUB_ROWS, SUB_ROWS)],
            out_hbm.at[cid, pl.ds(sid * SUB_ROWS, SUB_ROWS)],
        )

    return scat_kernel


def _tc_mm_body(x_ref, w_ref, o_ref):
    o_ref[...] = jnp.dot(x_ref[...], w_ref[...], precision=HIGHEST)


def _tc_mm(x, W1):
    return pl.pallas_call(
        _tc_mm_body,
        grid=(N // R,),
        in_specs=[
            pl.BlockSpec((R, D), lambda i: (i, 0)),
            pl.BlockSpec((D, H), lambda i: (0, 0)),
        ],
        out_specs=pl.BlockSpec((R, H), lambda i: (i, 0)),
        out_shape=jax.ShapeDtypeStruct((N, H), F32),
    )(x, W1)


def _tc_first_body(deg_ref, h_ref, hp_ref, dinv_ref):
    d = deg_ref[...]
    deg = d[0][:, :1] + d[1][:, :1] + 1.0
    dinv = lax.rsqrt(deg)
    hp_ref[...] = dinv * h_ref[...]
    dinv_ref[...] = jnp.broadcast_to(dinv, (R, 16))


def _tc_first(deg2, h1):
    return pl.pallas_call(
        _tc_first_body,
        grid=(N // R,),
        in_specs=[
            pl.BlockSpec((2, R, H), lambda i: (0, i, 0)),
            pl.BlockSpec((R, H), lambda i: (i, 0)),
        ],
        out_specs=[
            pl.BlockSpec((R, H), lambda i: (i, 0)),
            pl.BlockSpec((R, 16), lambda i: (i, 0)),
        ],
        out_shape=[
            jax.ShapeDtypeStruct((N, H), F32),
            jax.ShapeDtypeStruct((N, 16), F32),
        ],
    )(deg2, h1)


def _tc_mid_body(s_ref, hp_ref, dinv_ref, b_ref, w_ref, o_ref):
    dinv = dinv_ref[...][:, :1]
    s = s_ref[...]
    agg = s[0] + s[1] + hp_ref[...]
    a = jnp.maximum(dinv * agg + b_ref[...], 0.0)
    o_ref[...] = dinv * jnp.dot(a, w_ref[...], precision=HIGHEST)


def _tc_mid(S, hp, dinv16, b2d, W):
    return pl.pallas_call(
        _tc_mid_body,
        grid=(N // R,),
        in_specs=[
            pl.BlockSpec((2, R, H), lambda i: (0, i, 0)),
            pl.BlockSpec((R, H), lambda i: (i, 0)),
            pl.BlockSpec((R, 16), lambda i: (i, 0)),
            pl.BlockSpec((1, H), lambda i: (0, 0)),
            pl.BlockSpec((H, H), lambda i: (0, 0)),
        ],
        out_specs=pl.BlockSpec((R, H), lambda i: (i, 0)),
        out_shape=jax.ShapeDtypeStruct((N, H), F32),
    )(S, hp, dinv16, b2d, W)


def _tc_final_body(s_ref, hp_ref, dinv_ref, b_ref, batch_ref, wc_ref, bc_ref,
                   o_ref, pooled, cnt):
    i = pl.program_id(0)
    dinv = dinv_ref[...][:, :1]
    s = s_ref[...]
    a = jnp.maximum(dinv * (s[0] + s[1] + hp_ref[...]) + b_ref[...], 0.0)
    bb = batch_ref[0, 0, :]
    gid = lax.broadcasted_iota(jnp.int32, (G, R), 0)
    mask = jnp.where(gid == bb[None, :], 1.0, 0.0).astype(F32)
    pm = jnp.dot(mask, a, precision=HIGHEST)
    cm = jnp.broadcast_to(jnp.sum(mask, axis=1, keepdims=True), (G, H))

    @pl.when(i == 0)
    def _():
        pooled[...] = pm
        cnt[...] = cm

    @pl.when(i > 0)
    def _():
        pooled[...] += pm
        cnt[...] += cm

    @pl.when(i == N // R - 1)
    def _():
        emb = pooled[...] / jnp.maximum(cnt[...], 1.0)
        o_ref[...] = jnp.dot(emb, wc_ref[...], precision=HIGHEST) + bc_ref[...]


def _tc_final(S, hp, dinv16, b2d, batch3, Wcp, bcp):
    return pl.pallas_call(
        _tc_final_body,
        grid=(N // R,),
        in_specs=[
            pl.BlockSpec((2, R, H), lambda i: (0, i, 0)),
            pl.BlockSpec((R, H), lambda i: (i, 0)),
            pl.BlockSpec((R, 16), lambda i: (i, 0)),
            pl.BlockSpec((1, H), lambda i: (0, 0)),
            pl.BlockSpec((1, 1, R), lambda i: (i, 0, 0)),
            pl.BlockSpec((H, 128), lambda i: (0, 0)),
            pl.BlockSpec((1, 128), lambda i: (0, 0)),
        ],
        out_specs=pl.BlockSpec((G, 128), lambda i: (0, 0)),
        out_shape=jax.ShapeDtypeStruct((G, 128), F32),
        scratch_shapes=[
            pltpu.VMEM((G, H), F32),
            pltpu.VMEM((G, H), F32),
        ],
    )(S, hp, dinv16, b2d, batch3, Wcp, bcp)


def _split_uneven(v, fill, c0, c1):
    cmax = max(c0, c1)
    cap0 = SUBC * c0 * CHUNK
    cap1 = SUBC * c1 * CHUNK
    pad = cap0 + cap1 - v.shape[0]
    vp = jnp.concatenate([v, jnp.full((pad,), fill, jnp.int32)])
    p0 = vp[:cap0].reshape(SUBC, c0, CHUNK)
    p1 = vp[cap0:].reshape(SUBC, c1, CHUNK)
    p0 = jnp.pad(p0, ((0, 0), (0, cmax - c0), (0, 0)), constant_values=fill)
    p1 = jnp.pad(p1, ((0, 0), (0, cmax - c1), (0, 0)), constant_values=fill)
    return jnp.concatenate([p0, p1], axis=0)


def kernel(x, edge_index, batch, W1, b1, W2, b2, W3, b3, Wc, bc):
    src = edge_index[0].astype(jnp.int32)
    dst = edge_index[1].astype(jnp.int32)
    E = src.shape[0]
    ept = NTILES * CHUNK
    nchunks = -(-E // ept)
    e_pad = nchunks * ept
    pad = e_pad - E
    dstp = jnp.concatenate([dst, jnp.full((pad,), N, jnp.int32)]).reshape(
        NTILES, nchunks, CHUNK)
    tot = -(-E // CHUNK)
    c_slow = max(1, round(tot * SLOW_FRAC / SUBC))
    c_fast = -(-(tot - SUBC * c_slow) // SUBC)
    c0, c1 = (c_slow, c_fast) if SLOW_CORE == 0 else (c_fast, c_slow)
    srcp_s = _split_uneven(src, 0, c0, c1)
    dstp_s = _split_uneven(dst, N, c0, c1)

    onesH = jnp.ones((CHUNK, H), F32)
    zerosH = jnp.zeros((SUB_ROWS, H), F32)
    b1r = b1.reshape(1, H)
    b2r = b2.reshape(1, H)
    b3r = b3.reshape(1, H)
    Wcp = jnp.zeros((H, 128), F32).at[:, :C].set(Wc)
    bcp = jnp.zeros((1, 128), F32).at[0, :C].set(bc)
    batch3 = batch.astype(jnp.int32).reshape(N // R, 1, R)

    h1 = _tc_mm(x, W1)
    deg2 = _make_sc_degree(nchunks)(dstp, onesH, zerosH)
    h1p, dinv16 = _tc_first(deg2, h1)
    scat = _make_sc_scatter(c0, c1)
    S1 = scat(h1p, srcp_s, dstp_s, zerosH)
    h2p = _tc_mid(S1, h1p, dinv16, b1r, W2)
    S2 = scat(h2p, srcp_s, dstp_s, zerosH)
    h3p = _tc_mid(S2, h2p, dinv16, b2r, W3)
    S3 = scat(h3p, srcp_s, dstp_s, zerosH)
    logits = _tc_final(S3, h3p, dinv16, b3r, batch3, Wcp, bcp)
    return logits[:, :C]
